# baseline (speedup 1.0000x reference)
import functools

import jax
import jax.numpy as jnp
from jax import lax
from jax.experimental import pallas as pl
from jax.experimental.pallas import tpu as pltpu
from jax.experimental.pallas import tpu_sc as plsc

N = 10000
E = 320000
D = 128

NC = 2
NS = 16
NW = NC * NS

C = 128
CPW = 80
GP = NW * CPW
EP = GP * C
NP = 10240
ROWS_PER_TILE = NP // NS

_slope = 0.01

_GATHER_DNUMS = lax.GatherDimensionNumbers(
    offset_dims=(), collapsed_slice_dims=(0,), start_index_map=(0,))


def _lane_bcast(vec, lane):
    idx = jnp.full((16, 1), lane, jnp.int32)
    return lax.gather(vec, idx, dimension_numbers=_GATHER_DNUMS,
                      slice_sizes=(1,),
                      mode=lax.GatherScatterMode.PROMISE_IN_BOUNDS)


def _sc_agg_body(z_hbm, src_hbm, dst_hbm, ew_hbm, out_hbm,
                 src0, src1, dst0, dst1, ew0, ew1, rows0, rows1, acc_sh,
                 sem_i0, sem_i1, sem_g0, sem_g1):
    cid = lax.axis_index("c")
    sid = lax.axis_index("s")
    wid = sid * NC + cid
    base = wid * CPW

    srcs = (src0, src1)
    dsts = (dst0, dst1)
    ews = (ew0, ew1)
    rows = (rows0, rows1)
    sem_i = (sem_i0, sem_i1)
    sem_g = (sem_g0, sem_g1)

    def _idx_start(k, p):
        eb = (base + k) * C
        pltpu.async_copy(src_hbm.at[pl.ds(eb, C)], srcs[p], sem_i[p])
        pltpu.async_copy(dst_hbm.at[pl.ds(eb, C)], dsts[p], sem_i[p])
        pltpu.async_copy(ew_hbm.at[pl.ds(eb, C)], ews[p], sem_i[p])

    def _idx_wait(k, p):
        eb = (base + k) * C
        pltpu.make_async_copy(src_hbm.at[pl.ds(eb, C)], srcs[p], sem_i[p]).wait()
        pltpu.make_async_copy(dst_hbm.at[pl.ds(eb, C)], dsts[p], sem_i[p]).wait()
        pltpu.make_async_copy(ew_hbm.at[pl.ds(eb, C)], ews[p], sem_i[p]).wait()

    def _gather(k, p):
        pltpu.async_copy(z_hbm.at[srcs[p]], rows[p], sem_g[p])

    def _gwait(k, p):
        pltpu.make_async_copy(z_hbm.at[srcs[p]], rows[p], sem_g[p]).wait()

    def _scale(p):
        ew_v = ews[p]
        buf = rows[p]

        def _g(g, _):
            wvec = ew_v[pl.ds(g * 16, 16)]
            for lane in range(16):
                bw = _lane_bcast(wvec, lane)
                e = g * 16 + lane
                for j in range(D // 16):
                    sl = pl.ds(j * 16, 16)
                    buf[e, sl] = buf[e, sl] * bw
            return 0
        lax.fori_loop(0, C // 16, _g, 0)

    def _scatter(p):
        pltpu.sync_copy(rows[p], acc_sh.at[dsts[p]], add=True)

    _idx_start(0, 0)
    _idx_start(1, 1)

    def _zrow(i, _):
        for j in range(D // 16):
            rows0[i, pl.ds(j * 16, 16)] = jnp.zeros((16,), jnp.float32)
        return 0
    lax.fori_loop(0, C, _zrow, 0)
    zbase = sid * ROWS_PER_TILE
    for kz in range(ROWS_PER_TILE // C):
        pltpu.sync_copy(rows0, acc_sh.at[pl.ds(zbase + kz * C, C)])
    plsc.subcore_barrier()

    _idx_wait(0, 0)
    _gather(0, 0)

    def _pipe(k, par):
        _idx_wait(k + 1, 1 - par)
        _gather(k + 1, 1 - par)
        _gwait(k, par)
        _scale(par)
        _scatter(par)

        @pl.when(k < CPW - 2)
        def _():
            _idx_start(k + 2, par)

    def _pipe2(i2, _):
        _pipe(2 * i2, 0)
        _pipe(2 * i2 + 1, 1)
        return 0
    lax.fori_loop(0, (CPW - 1) // 2, _pipe2, 0)
    _pipe(CPW - 2, 0)
    _gwait(CPW - 1, 1)
    _scale(1)
    _scatter(1)

    plsc.subcore_barrier()

    pltpu.sync_copy(acc_sh.at[pl.ds(zbase, ROWS_PER_TILE)],
                    out_hbm.at[cid, pl.ds(zbase, ROWS_PER_TILE)])


@functools.partial(jax.jit, static_argnames=())
def _sc_agg(z, src, dst, ew):
    mesh = plsc.VectorSubcoreMesh(core_axis_name="c", subcore_axis_name="s")
    f = pl.kernel(
        _sc_agg_body,
        out_type=jax.ShapeDtypeStruct((NC, NP, D), jnp.float32),
        mesh=mesh,
        scratch_types=[
            pltpu.VMEM((C,), jnp.int32),
            pltpu.VMEM((C,), jnp.int32),
            pltpu.VMEM((C,), jnp.int32),
            pltpu.VMEM((C,), jnp.int32),
            pltpu.VMEM((C,), jnp.float32),
            pltpu.VMEM((C,), jnp.float32),
            pltpu.VMEM((C, D), jnp.float32),
            pltpu.VMEM((C, D), jnp.float32),
            pltpu.VMEM_SHARED((NP, D), jnp.float32),
            pltpu.SemaphoreType.DMA,
            pltpu.SemaphoreType.DMA,
            pltpu.SemaphoreType.DMA,
            pltpu.SemaphoreType.DMA,
        ],
    )
    return f(z, src, dst, ew)


_BN = 1000


def _tc_pre_body(x_ref, w0t_ref, w1t_ref, b_ref, a_ref, z_ref):
    x = x_ref[...]
    a_ref[...] = (jnp.dot(x, w0t_ref[...], preferred_element_type=jnp.float32,
                          precision=lax.Precision.HIGHEST)
                  + b_ref[...] + x)
    z_ref[...] = jnp.dot(x, w1t_ref[...], preferred_element_type=jnp.float32,
                         precision=lax.Precision.HIGHEST)


def _tc_pre(x, w0t, w1t, b):
    grid = (N // _BN,)
    return pl.pallas_call(
        _tc_pre_body,
        grid=grid,
        in_specs=[
            pl.BlockSpec((_BN, D), lambda i: (i, 0)),
            pl.BlockSpec((D, D), lambda i: (0, 0)),
            pl.BlockSpec((D, D), lambda i: (0, 0)),
            pl.BlockSpec((1, D), lambda i: (0, 0)),
        ],
        out_specs=[
            pl.BlockSpec((_BN, D), lambda i: (i, 0)),
            pl.BlockSpec((_BN, D), lambda i: (i, 0)),
        ],
        out_shape=[
            jax.ShapeDtypeStruct((N, D), jnp.float32),
            jax.ShapeDtypeStruct((N, D), jnp.float32),
        ],
    )(x, w0t, w1t, b)


def _tc_mid_body(a_ref, agg_ref, w0t_ref, w1t_ref, b_ref, a_out_ref, z_out_ref):
    h = a_ref[...] + agg_ref[0] + agg_ref[1]
    y = jnp.where(h >= 0, h, _slope * h)
    a_out_ref[...] = (jnp.dot(y, w0t_ref[...], preferred_element_type=jnp.float32,
                              precision=lax.Precision.HIGHEST)
                      + b_ref[...] + y)
    z_out_ref[...] = jnp.dot(y, w1t_ref[...], preferred_element_type=jnp.float32,
                             precision=lax.Precision.HIGHEST)


def _tc_mid(a, agg, w0t, w1t, b):
    grid = (N // _BN,)
    return pl.pallas_call(
        _tc_mid_body,
        grid=grid,
        in_specs=[
            pl.BlockSpec((_BN, D), lambda i: (i, 0)),
            pl.BlockSpec((NC, _BN, D), lambda i: (0, i, 0)),
            pl.BlockSpec((D, D), lambda i: (0, 0)),
            pl.BlockSpec((D, D), lambda i: (0, 0)),
            pl.BlockSpec((1, D), lambda i: (0, 0)),
        ],
        out_specs=[
            pl.BlockSpec((_BN, D), lambda i: (i, 0)),
            pl.BlockSpec((_BN, D), lambda i: (i, 0)),
        ],
        out_shape=[
            jax.ShapeDtypeStruct((N, D), jnp.float32),
            jax.ShapeDtypeStruct((N, D), jnp.float32),
        ],
    )(a, agg, w0t, w1t, b)


def _tc_post_body(a_ref, agg_ref, y_ref):
    h = a_ref[...] + agg_ref[0] + agg_ref[1]
    y_ref[...] = jnp.where(h >= 0, h, _slope * h)


def _tc_post(a, agg):
    grid = (N // _BN,)
    return pl.pallas_call(
        _tc_post_body,
        grid=grid,
        in_specs=[
            pl.BlockSpec((_BN, D), lambda i: (i, 0)),
            pl.BlockSpec((NC, _BN, D), lambda i: (0, i, 0)),
        ],
        out_specs=pl.BlockSpec((_BN, D), lambda i: (i, 0)),
        out_shape=jax.ShapeDtypeStruct((N, D), jnp.float32),
    )(a, agg)


def kernel(y, edge_index, edge_weight, W0_0, W1_0, b_0, W0_1, W1_1, b_1):
    pad = EP - E
    src = jnp.concatenate([edge_index[0], jnp.zeros((pad,), jnp.int32)])
    dst = jnp.concatenate([edge_index[1], jnp.zeros((pad,), jnp.int32)])
    ew = jnp.concatenate([edge_weight, jnp.zeros((pad,), jnp.float32)])
    a0, z0 = _tc_pre(y, W0_0.T, W1_0.T, b_0.reshape(1, D))
    agg0 = _sc_agg(z0, src, dst, ew)
    a1, z1 = _tc_mid(a0, agg0, W0_1.T, W1_1.T, b_1.reshape(1, D))
    agg1 = _sc_agg(z1, src, dst, ew)
    return _tc_post(a1, agg1)


# speedup vs baseline: 3.1167x; 3.1167x over previous
import functools

import jax
import jax.numpy as jnp
from jax import lax
from jax.experimental import pallas as pl
from jax.experimental.pallas import tpu as pltpu
from jax.experimental.pallas import tpu_sc as plsc

N = 10000
E = 320000
D = 128

NC = 2
NS = 16
NW = NC * NS

C = 128
CPW = 80
GP = NW * CPW
EP = GP * C
NP = 10240
ROWS_PER_TILE = NP // NS

_slope = 0.01

_GATHER_DNUMS = lax.GatherDimensionNumbers(
    offset_dims=(), collapsed_slice_dims=(0,), start_index_map=(0,))


def _lane_bcast(vec, lane):
    idx = jnp.full((16, 1), lane, jnp.int32)
    return lax.gather(vec, idx, dimension_numbers=_GATHER_DNUMS,
                      slice_sizes=(1,),
                      mode=lax.GatherScatterMode.PROMISE_IN_BOUNDS)


def _sc_agg_body(z_hbm, src_hbm, dst_hbm, ew_hbm, out_hbm,
                 src0, src1, src2, src3, dst0, dst1, dst2, dst3,
                 ew0, ew1, ew2, ew3, rows0, rows1, acc_sh,
                 sem_i0, sem_i1, sem_i2, sem_i3,
                 sem_g0, sem_g1, sem_s0, sem_s1):
    cid = lax.axis_index("c")
    sid = lax.axis_index("s")
    wid = sid * NC + cid
    base = wid * CPW

    srcs = (src0, src1, src2, src3)
    dsts = (dst0, dst1, dst2, dst3)
    ews = (ew0, ew1, ew2, ew3)
    rows = (rows0, rows1)
    sem_i = (sem_i0, sem_i1, sem_i2, sem_i3)
    sem_g = (sem_g0, sem_g1)
    sem_s = (sem_s0, sem_s1)

    def _idx_start(k, p4):
        eb = (base + k) * C
        pltpu.async_copy(src_hbm.at[pl.ds(eb, C)], srcs[p4], sem_i[p4])
        pltpu.async_copy(dst_hbm.at[pl.ds(eb, C)], dsts[p4], sem_i[p4])
        pltpu.async_copy(ew_hbm.at[pl.ds(eb, C)], ews[p4], sem_i[p4])

    def _idx_wait(k, p4):
        eb = (base + k) * C
        pltpu.make_async_copy(src_hbm.at[pl.ds(eb, C)], srcs[p4], sem_i[p4]).wait()
        pltpu.make_async_copy(dst_hbm.at[pl.ds(eb, C)], dsts[p4], sem_i[p4]).wait()
        pltpu.make_async_copy(ew_hbm.at[pl.ds(eb, C)], ews[p4], sem_i[p4]).wait()

    def _gather(p2, p4):
        pltpu.async_copy(z_hbm.at[srcs[p4]], rows[p2], sem_g[p2])

    def _gwait(p2):
        pltpu.make_async_copy(z_hbm.at[srcs[0]], rows[p2], sem_g[p2]).wait()

    def _scale(p2, p4):
        ew_v = ews[p4]
        buf = rows[p2]

        def _g(g, _):
            wvec = ew_v[pl.ds(g * 16, 16)]
            for lane in range(16):
                bw = _lane_bcast(wvec, lane)
                e = g * 16 + lane
                for j in range(D // 16):
                    sl = pl.ds(j * 16, 16)
                    buf[e, sl] = buf[e, sl] * bw
            return 0
        lax.fori_loop(0, C // 16, _g, 0)

    def _sstart(p2, p4):
        pltpu.async_copy(rows[p2], acc_sh.at[dsts[p4]], sem_s[p2], add=True)

    def _swait(p2):
        pltpu.make_async_copy(rows[p2], acc_sh.at[dsts[0]], sem_s[p2]).wait()

    _idx_start(0, 0)
    _idx_start(1, 1)

    def _zrow(i, _):
        for j in range(D // 16):
            rows0[i, pl.ds(j * 16, 16)] = jnp.zeros((16,), jnp.float32)
        return 0
    lax.fori_loop(0, C, _zrow, 0)
    zbase = sid * ROWS_PER_TILE
    for kz in range(ROWS_PER_TILE // C):
        pltpu.sync_copy(rows0, acc_sh.at[pl.ds(zbase + kz * C, C)])
    plsc.subcore_barrier()

    _idx_wait(0, 0)
    _gather(0, 0)

    def _pipe(k, p2, p4):
        _idx_wait(k + 1, (p4 + 1) % 4)

        @pl.when(k >= 1)
        def _():
            _swait(1 - p2)
        _gather(1 - p2, (p4 + 1) % 4)
        _gwait(p2)
        _scale(p2, p4)
        _sstart(p2, p4)

        @pl.when(k < CPW - 2)
        def _():
            _idx_start(k + 2, (p4 + 2) % 4)

    def _pipe4(i4, _):
        k0 = 4 * i4
        _pipe(k0, 0, 0)
        _pipe(k0 + 1, 1, 1)
        _pipe(k0 + 2, 0, 2)
        _pipe(k0 + 3, 1, 3)
        return 0
    lax.fori_loop(0, CPW // 4 - 1, _pipe4, 0)
    _pipe(CPW - 4, 0, 0)
    _pipe(CPW - 3, 1, 1)
    _pipe(CPW - 2, 0, 2)
    _gwait(1)
    _scale(1, 3)
    _swait(0)
    _sstart(1, 3)
    _swait(1)

    plsc.subcore_barrier()

    pltpu.sync_copy(acc_sh.at[pl.ds(zbase, ROWS_PER_TILE)],
                    out_hbm.at[cid, pl.ds(zbase, ROWS_PER_TILE)])


@functools.partial(jax.jit, static_argnames=())
def _sc_agg(z, src, dst, ew):
    mesh = plsc.VectorSubcoreMesh(core_axis_name="c", subcore_axis_name="s")
    f = pl.kernel(
        _sc_agg_body,
        out_type=jax.ShapeDtypeStruct((NC, NP, D), jnp.float32),
        mesh=mesh,
        scratch_types=(
            [pltpu.VMEM((C,), jnp.int32)] * 8
            + [pltpu.VMEM((C,), jnp.float32)] * 4
            + [pltpu.VMEM((C, D), jnp.float32)] * 2
            + [pltpu.VMEM_SHARED((NP, D), jnp.float32)]
            + [pltpu.SemaphoreType.DMA] * 8
        ),
    )
    return f(z, src, dst, ew)


_BN = 1000


def _tc_pre_body(x_ref, w0t_ref, w1t_ref, b_ref, a_ref, z_ref):
    x = x_ref[...]
    a_ref[...] = (jnp.dot(x, w0t_ref[...], preferred_element_type=jnp.float32,
                          precision=lax.Precision.HIGHEST)
                  + b_ref[...] + x)
    z_ref[...] = jnp.dot(x, w1t_ref[...], preferred_element_type=jnp.float32,
                         precision=lax.Precision.HIGHEST)


def _tc_pre(x, w0t, w1t, b):
    grid = (N // _BN,)
    return pl.pallas_call(
        _tc_pre_body,
        grid=grid,
        in_specs=[
            pl.BlockSpec((_BN, D), lambda i: (i, 0)),
            pl.BlockSpec((D, D), lambda i: (0, 0)),
            pl.BlockSpec((D, D), lambda i: (0, 0)),
            pl.BlockSpec((1, D), lambda i: (0, 0)),
        ],
        out_specs=[
            pl.BlockSpec((_BN, D), lambda i: (i, 0)),
            pl.BlockSpec((_BN, D), lambda i: (i, 0)),
        ],
        out_shape=[
            jax.ShapeDtypeStruct((N, D), jnp.float32),
            jax.ShapeDtypeStruct((N, D), jnp.float32),
        ],
    )(x, w0t, w1t, b)


def _tc_mid_body(a_ref, agg_ref, w0t_ref, w1t_ref, b_ref, a_out_ref, z_out_ref):
    h = a_ref[...] + agg_ref[0] + agg_ref[1]
    y = jnp.where(h >= 0, h, _slope * h)
    a_out_ref[...] = (jnp.dot(y, w0t_ref[...], preferred_element_type=jnp.float32,
                              precision=lax.Precision.HIGHEST)
                      + b_ref[...] + y)
    z_out_ref[...] = jnp.dot(y, w1t_ref[...], preferred_element_type=jnp.float32,
                             precision=lax.Precision.HIGHEST)


def _tc_mid(a, agg, w0t, w1t, b):
    grid = (N // _BN,)
    return pl.pallas_call(
        _tc_mid_body,
        grid=grid,
        in_specs=[
            pl.BlockSpec((_BN, D), lambda i: (i, 0)),
            pl.BlockSpec((NC, _BN, D), lambda i: (0, i, 0)),
            pl.BlockSpec((D, D), lambda i: (0, 0)),
            pl.BlockSpec((D, D), lambda i: (0, 0)),
            pl.BlockSpec((1, D), lambda i: (0, 0)),
        ],
        out_specs=[
            pl.BlockSpec((_BN, D), lambda i: (i, 0)),
            pl.BlockSpec((_BN, D), lambda i: (i, 0)),
        ],
        out_shape=[
            jax.ShapeDtypeStruct((N, D), jnp.float32),
            jax.ShapeDtypeStruct((N, D), jnp.float32),
        ],
    )(a, agg, w0t, w1t, b)


def _tc_post_body(a_ref, agg_ref, y_ref):
    h = a_ref[...] + agg_ref[0] + agg_ref[1]
    y_ref[...] = jnp.where(h >= 0, h, _slope * h)


def _tc_post(a, agg):
    grid = (N // _BN,)
    return pl.pallas_call(
        _tc_post_body,
        grid=grid,
        in_specs=[
            pl.BlockSpec((_BN, D), lambda i: (i, 0)),
            pl.BlockSpec((NC, _BN, D), lambda i: (0, i, 0)),
        ],
        out_specs=pl.BlockSpec((_BN, D), lambda i: (i, 0)),
        out_shape=jax.ShapeDtypeStruct((N, D), jnp.float32),
    )(a, agg)


def kernel(y, edge_index, edge_weight, W0_0, W1_0, b_0, W0_1, W1_1, b_1):
    pad = EP - E
    src = jnp.concatenate([edge_index[0], jnp.zeros((pad,), jnp.int32)])
    dst = jnp.concatenate([edge_index[1], jnp.zeros((pad,), jnp.int32)])
    ew = jnp.concatenate([edge_weight, jnp.zeros((pad,), jnp.float32)])
    a0, z0 = _tc_pre(y, W0_0.T, W1_0.T, b_0.reshape(1, D))
    agg0 = _sc_agg(z0, src, dst, ew)
    a1, z1 = _tc_mid(a0, agg0, W0_1.T, W1_1.T, b_1.reshape(1, D))
    agg1 = _sc_agg(z1, src, dst, ew)
    return _tc_post(a1, agg1)


# speedup vs baseline: 3.2937x; 1.0568x over previous
import functools

import jax
import jax.numpy as jnp
from jax import lax
from jax.experimental import pallas as pl
from jax.experimental.pallas import tpu as pltpu
from jax.experimental.pallas import tpu_sc as plsc

N = 10000
E = 320000
D = 128

NC = 2
NS = 16
NW = NC * NS

C = 128
CPW = 80
GP = NW * CPW
EP = GP * C
NP = 10240
ROWS_PER_TILE = NP // NS

_slope = 0.01

_GATHER_DNUMS = lax.GatherDimensionNumbers(
    offset_dims=(), collapsed_slice_dims=(0,), start_index_map=(0,))


def _lane_bcast(vec, lane):
    idx = jnp.full((16, 1), lane, jnp.int32)
    return lax.gather(vec, idx, dimension_numbers=_GATHER_DNUMS,
                      slice_sizes=(1,),
                      mode=lax.GatherScatterMode.PROMISE_IN_BOUNDS)


def _sc_agg_body(z_hbm, src_hbm, dst_hbm, ew_hbm, out_hbm,
                 src0, src1, src2, src3, dst0, dst1, dst2, dst3,
                 ew0, ew1, ew2, ew3, rows0, rows1, acc_sh,
                 sem_i0, sem_i1, sem_i2, sem_i3,
                 sem_g0, sem_g1, sem_s0, sem_s1):
    cid = lax.axis_index("c")
    sid = lax.axis_index("s")
    wid = sid * NC + cid
    base = wid * CPW

    srcs = (src0, src1, src2, src3)
    dsts = (dst0, dst1, dst2, dst3)
    ews = (ew0, ew1, ew2, ew3)
    rows = (rows0, rows1)
    sem_i = (sem_i0, sem_i1, sem_i2, sem_i3)
    sem_g = (sem_g0, sem_g1)
    sem_s = (sem_s0, sem_s1)

    def _idx_start(k, p4):
        eb = (base + k) * C
        pltpu.async_copy(src_hbm.at[pl.ds(eb, C)], srcs[p4], sem_i[p4])
        pltpu.async_copy(dst_hbm.at[pl.ds(eb, C)], dsts[p4], sem_i[p4])
        pltpu.async_copy(ew_hbm.at[pl.ds(eb, C)], ews[p4], sem_i[p4])

    def _idx_wait(k, p4):
        eb = (base + k) * C
        pltpu.make_async_copy(src_hbm.at[pl.ds(eb, C)], srcs[p4], sem_i[p4]).wait()
        pltpu.make_async_copy(dst_hbm.at[pl.ds(eb, C)], dsts[p4], sem_i[p4]).wait()
        pltpu.make_async_copy(ew_hbm.at[pl.ds(eb, C)], ews[p4], sem_i[p4]).wait()

    def _gather(p2, p4):
        pltpu.async_copy(z_hbm.at[srcs[p4]], rows[p2], sem_g[p2])

    def _gwait(p2):
        pltpu.make_async_copy(z_hbm.at[srcs[0]], rows[p2], sem_g[p2]).wait()

    def _scale(p2, p4):
        ew_v = ews[p4]
        buf = rows[p2]

        def _g(g, _):
            wvec = ew_v[pl.ds(g * 16, 16)]
            for lane in range(16):
                bw = _lane_bcast(wvec, lane)
                e = g * 16 + lane
                for j in range(D // 16):
                    sl = pl.ds(j * 16, 16)
                    buf[e, sl] = buf[e, sl] * bw
            return 0
        lax.fori_loop(0, C // 16, _g, 0)

    def _sstart(p2, p4):
        pltpu.async_copy(rows[p2], acc_sh.at[dsts[p4]], sem_s[p2], add=True)

    def _swait(p2):
        pltpu.make_async_copy(rows[p2], acc_sh.at[dsts[0]], sem_s[p2]).wait()

    _idx_start(0, 0)
    _idx_start(1, 1)

    def _zrow(i, _):
        for j in range(D // 16):
            rows0[i, pl.ds(j * 16, 16)] = jnp.zeros((16,), jnp.float32)
        return 0
    lax.fori_loop(0, C, _zrow, 0)
    zbase = sid * ROWS_PER_TILE
    for kz in range(ROWS_PER_TILE // C):
        pltpu.sync_copy(rows0, acc_sh.at[pl.ds(zbase + kz * C, C)])
    plsc.subcore_barrier()

    _idx_wait(0, 0)
    _gather(0, 0)

    def _pipe(k, p2, p4):
        _idx_wait(k + 1, (p4 + 1) % 4)

        @pl.when(k >= 1)
        def _():
            _swait(1 - p2)
        _gather(1 - p2, (p4 + 1) % 4)
        _gwait(p2)
        _scale(p2, p4)
        _sstart(p2, p4)

        @pl.when(k < CPW - 2)
        def _():
            _idx_start(k + 2, (p4 + 2) % 4)

    def _pipe4(i4, _):
        k0 = 4 * i4
        _pipe(k0, 0, 0)
        _pipe(k0 + 1, 1, 1)
        _pipe(k0 + 2, 0, 2)
        _pipe(k0 + 3, 1, 3)
        return 0
    lax.fori_loop(0, CPW // 4 - 1, _pipe4, 0)
    _pipe(CPW - 4, 0, 0)
    _pipe(CPW - 3, 1, 1)
    _pipe(CPW - 2, 0, 2)
    _gwait(1)
    _scale(1, 3)
    _swait(0)
    _sstart(1, 3)
    _swait(1)

    plsc.subcore_barrier()

    pltpu.sync_copy(acc_sh.at[pl.ds(zbase, ROWS_PER_TILE)],
                    out_hbm.at[cid, pl.ds(zbase, ROWS_PER_TILE)])


@functools.partial(jax.jit, static_argnames=())
def _sc_agg(z, src, dst, ew):
    mesh = plsc.VectorSubcoreMesh(core_axis_name="c", subcore_axis_name="s")
    f = pl.kernel(
        _sc_agg_body,
        out_type=jax.ShapeDtypeStruct((NC, NP, D), jnp.float32),
        mesh=mesh,
        scratch_types=(
            [pltpu.VMEM((C,), jnp.int32)] * 8
            + [pltpu.VMEM((C,), jnp.float32)] * 4
            + [pltpu.VMEM((C, D), jnp.float32)] * 2
            + [pltpu.VMEM_SHARED((NP, D), jnp.float32)]
            + [pltpu.SemaphoreType.DMA] * 8
        ),
    )
    return f(z, src, dst, ew)


_BN = 1000


def _dot_t(x, w):
    return lax.dot_general(x, w, (((1,), (1,)), ((), ())),
                           preferred_element_type=jnp.float32)


def _tc_pre_body(x_ref, w0t_ref, w1t_ref, b_ref, a_ref, z_ref):
    x = x_ref[...]
    a_ref[...] = _dot_t(x, w0t_ref[...]) + b_ref[...] + x
    z_ref[...] = _dot_t(x, w1t_ref[...])


def _tc_pre(x, w0t, w1t, b):
    grid = (N // _BN,)
    return pl.pallas_call(
        _tc_pre_body,
        grid=grid,
        in_specs=[
            pl.BlockSpec((_BN, D), lambda i: (i, 0)),
            pl.BlockSpec((D, D), lambda i: (0, 0)),
            pl.BlockSpec((D, D), lambda i: (0, 0)),
            pl.BlockSpec((1, D), lambda i: (0, 0)),
        ],
        out_specs=[
            pl.BlockSpec((_BN, D), lambda i: (i, 0)),
            pl.BlockSpec((_BN, D), lambda i: (i, 0)),
        ],
        out_shape=[
            jax.ShapeDtypeStruct((N, D), jnp.float32),
            jax.ShapeDtypeStruct((N, D), jnp.float32),
        ],
    )(x, w0t, w1t, b)


def _tc_mid_body(a_ref, agg_ref, w0t_ref, w1t_ref, b_ref, a_out_ref, z_out_ref):
    h = a_ref[...] + agg_ref[0] + agg_ref[1]
    y = jnp.where(h >= 0, h, _slope * h)
    a_out_ref[...] = _dot_t(y, w0t_ref[...]) + b_ref[...] + y
    z_out_ref[...] = _dot_t(y, w1t_ref[...])


def _tc_mid(a, agg, w0t, w1t, b):
    grid = (N // _BN,)
    return pl.pallas_call(
        _tc_mid_body,
        grid=grid,
        in_specs=[
            pl.BlockSpec((_BN, D), lambda i: (i, 0)),
            pl.BlockSpec((NC, _BN, D), lambda i: (0, i, 0)),
            pl.BlockSpec((D, D), lambda i: (0, 0)),
            pl.BlockSpec((D, D), lambda i: (0, 0)),
            pl.BlockSpec((1, D), lambda i: (0, 0)),
        ],
        out_specs=[
            pl.BlockSpec((_BN, D), lambda i: (i, 0)),
            pl.BlockSpec((_BN, D), lambda i: (i, 0)),
        ],
        out_shape=[
            jax.ShapeDtypeStruct((N, D), jnp.float32),
            jax.ShapeDtypeStruct((N, D), jnp.float32),
        ],
    )(a, agg, w0t, w1t, b)


def _tc_post_body(a_ref, agg_ref, y_ref):
    h = a_ref[...] + agg_ref[0] + agg_ref[1]
    y_ref[...] = jnp.where(h >= 0, h, _slope * h)


def _tc_post(a, agg):
    grid = (N // _BN,)
    return pl.pallas_call(
        _tc_post_body,
        grid=grid,
        in_specs=[
            pl.BlockSpec((_BN, D), lambda i: (i, 0)),
            pl.BlockSpec((NC, _BN, D), lambda i: (0, i, 0)),
        ],
        out_specs=pl.BlockSpec((_BN, D), lambda i: (i, 0)),
        out_shape=jax.ShapeDtypeStruct((N, D), jnp.float32),
    )(a, agg)


def kernel(y, edge_index, edge_weight, W0_0, W1_0, b_0, W0_1, W1_1, b_1):
    pad = EP - E
    spread = jnp.arange(pad, dtype=jnp.int32) % N
    src = jnp.concatenate([edge_index[0], spread])
    dst = jnp.concatenate([edge_index[1], spread])
    ew = jnp.concatenate([edge_weight, jnp.zeros((pad,), jnp.float32)])
    a0, z0 = _tc_pre(y, W0_0, W1_0, b_0.reshape(1, D))
    agg0 = _sc_agg(z0, src, dst, ew)
    a1, z1 = _tc_mid(a0, agg0, W0_1, W1_1, b_1.reshape(1, D))
    agg1 = _sc_agg(z1, src, dst, ew)
    return _tc_post(a1, agg1)


# speedup vs baseline: 3.4296x; 1.0412x over previous
import functools

import jax
import jax.numpy as jnp
from jax import lax
from jax.experimental import pallas as pl
from jax.experimental.pallas import tpu as pltpu
from jax.experimental.pallas import tpu_sc as plsc

N = 10000
E = 320000
D = 128

NC = 2
NS = 16
NW = NC * NS

C = 128
G = E // C
CPW = G // NW
NEXTRA = G - NW * CPW
NP = 10240
ROWS_PER_TILE = NP // NS

_slope = 0.01

_GATHER_DNUMS = lax.GatherDimensionNumbers(
    offset_dims=(), collapsed_slice_dims=(0,), start_index_map=(0,))


def _lane_bcast(vec, lane):
    idx = jnp.full((16, 1), lane, jnp.int32)
    return lax.gather(vec, idx, dimension_numbers=_GATHER_DNUMS,
                      slice_sizes=(1,),
                      mode=lax.GatherScatterMode.PROMISE_IN_BOUNDS)


def _sc_agg_body(z_hbm, ei_hbm, ew_hbm, out_hbm,
                 src0, src1, src2, src3, dst0, dst1, dst2, dst3,
                 ew0, ew1, ew2, ew3, rows0, rows1, acc_sh,
                 sem_i0, sem_i1, sem_i2, sem_i3,
                 sem_g0, sem_g1, sem_s0, sem_s1):
    cid = lax.axis_index("c")
    sid = lax.axis_index("s")
    wid = sid * NC + cid

    srcs = (src0, src1, src2, src3)
    dsts = (dst0, dst1, dst2, dst3)
    ews = (ew0, ew1, ew2, ew3)
    rows = (rows0, rows1)
    sem_i = (sem_i0, sem_i1, sem_i2, sem_i3)
    sem_g = (sem_g0, sem_g1)
    sem_s = (sem_s0, sem_s1)

    def _eb(k):
        return (wid + k * NW) * C

    def _idx_start(k, p4):
        eb = _eb(k)
        pltpu.async_copy(ei_hbm.at[0, pl.ds(eb, C)], srcs[p4], sem_i[p4])
        pltpu.async_copy(ei_hbm.at[1, pl.ds(eb, C)], dsts[p4], sem_i[p4])
        pltpu.async_copy(ew_hbm.at[pl.ds(eb, C)], ews[p4], sem_i[p4])

    def _idx_wait(k, p4):
        eb = _eb(k)
        pltpu.make_async_copy(ei_hbm.at[0, pl.ds(eb, C)], srcs[p4], sem_i[p4]).wait()
        pltpu.make_async_copy(ei_hbm.at[1, pl.ds(eb, C)], dsts[p4], sem_i[p4]).wait()
        pltpu.make_async_copy(ew_hbm.at[pl.ds(eb, C)], ews[p4], sem_i[p4]).wait()

    def _gather(p2, p4):
        pltpu.async_copy(z_hbm.at[srcs[p4]], rows[p2], sem_g[p2])

    def _gwait(p2):
        pltpu.make_async_copy(z_hbm.at[srcs[0]], rows[p2], sem_g[p2]).wait()

    def _scale(p2, p4):
        ew_v = ews[p4]
        buf = rows[p2]

        def _g(g, _):
            wvec = ew_v[pl.ds(g * 16, 16)]
            for lane in range(16):
                bw = _lane_bcast(wvec, lane)
                e = g * 16 + lane
                for j in range(D // 16):
                    sl = pl.ds(j * 16, 16)
                    buf[e, sl] = buf[e, sl] * bw
            return 0
        lax.fori_loop(0, C // 16, _g, 0)

    def _sstart(p2, p4):
        pltpu.async_copy(rows[p2], acc_sh.at[dsts[p4]], sem_s[p2], add=True)

    def _swait(p2):
        pltpu.make_async_copy(rows[p2], acc_sh.at[dsts[0]], sem_s[p2]).wait()

    _idx_start(0, 0)
    _idx_start(1, 1)

    def _zrow(i, _):
        for j in range(D // 16):
            rows0[i, pl.ds(j * 16, 16)] = jnp.zeros((16,), jnp.float32)
        return 0
    lax.fori_loop(0, C, _zrow, 0)
    zbase = sid * ROWS_PER_TILE
    for kz in range(ROWS_PER_TILE // C):
        pltpu.sync_copy(rows0, acc_sh.at[pl.ds(zbase + kz * C, C)])
    plsc.subcore_barrier()

    _idx_wait(0, 0)
    _gather(0, 0)

    def _pipe(k, p2, p4):
        k = jnp.asarray(k, jnp.int32)
        _idx_wait(k + 1, (p4 + 1) % 4)

        @pl.when(k >= 1)
        def _():
            _swait(1 - p2)
        _gather(1 - p2, (p4 + 1) % 4)
        _gwait(p2)
        _scale(p2, p4)
        _sstart(p2, p4)

        @pl.when(k < CPW - 2)
        def _():
            _idx_start(k + 2, (p4 + 2) % 4)

    def _pipe4(i4, _):
        k0 = 4 * i4
        _pipe(k0, 0, 0)
        _pipe(k0 + 1, 1, 1)
        _pipe(k0 + 2, 0, 2)
        _pipe(k0 + 3, 1, 3)
        return 0
    lax.fori_loop(0, CPW // 4 - 1, _pipe4, 0)
    for k in range(4 * (CPW // 4 - 1), CPW - 1):
        _pipe(k, k % 2, k % 4)
    lastp2 = (CPW - 1) % 2
    lastp4 = (CPW - 1) % 4
    _gwait(lastp2)
    _scale(lastp2, lastp4)
    _swait(1 - lastp2)
    _sstart(lastp2, lastp4)
    _swait(lastp2)

    @pl.when(wid < NEXTRA)
    def _():
        eb = (NW * CPW + wid) * C
        pltpu.sync_copy(ei_hbm.at[0, pl.ds(eb, C)], srcs[0])
        pltpu.sync_copy(ei_hbm.at[1, pl.ds(eb, C)], dsts[0])
        pltpu.sync_copy(ew_hbm.at[pl.ds(eb, C)], ews[0])
        pltpu.async_copy(z_hbm.at[srcs[0]], rows[0], sem_g[0]).wait()
        _scale(0, 0)
        pltpu.sync_copy(rows[0], acc_sh.at[dsts[0]], add=True)

    plsc.subcore_barrier()

    pltpu.sync_copy(acc_sh.at[pl.ds(zbase, ROWS_PER_TILE)],
                    out_hbm.at[cid, pl.ds(zbase, ROWS_PER_TILE)])


@functools.partial(jax.jit, static_argnames=())
def _sc_agg(z, ei, ew):
    mesh = plsc.VectorSubcoreMesh(core_axis_name="c", subcore_axis_name="s")
    f = pl.kernel(
        _sc_agg_body,
        out_type=jax.ShapeDtypeStruct((NC, NP, D), jnp.float32),
        mesh=mesh,
        scratch_types=(
            [pltpu.VMEM((C,), jnp.int32)] * 8
            + [pltpu.VMEM((C,), jnp.float32)] * 4
            + [pltpu.VMEM((C, D), jnp.float32)] * 2
            + [pltpu.VMEM_SHARED((NP, D), jnp.float32)]
            + [pltpu.SemaphoreType.DMA] * 8
        ),
    )
    return f(z, ei, ew)


_BN = 1000


def _dot_t(x, w):
    return lax.dot_general(x, w, (((1,), (1,)), ((), ())),
                           preferred_element_type=jnp.float32)


def _tc_pre_body(x_ref, w0t_ref, w1t_ref, b_ref, a_ref, z_ref):
    x = x_ref[...]
    a_ref[...] = _dot_t(x, w0t_ref[...]) + b_ref[...] + x
    z_ref[...] = _dot_t(x, w1t_ref[...])


def _tc_pre(x, w0t, w1t, b):
    grid = (N // _BN,)
    return pl.pallas_call(
        _tc_pre_body,
        grid=grid,
        in_specs=[
            pl.BlockSpec((_BN, D), lambda i: (i, 0)),
            pl.BlockSpec((D, D), lambda i: (0, 0)),
            pl.BlockSpec((D, D), lambda i: (0, 0)),
            pl.BlockSpec((1, D), lambda i: (0, 0)),
        ],
        out_specs=[
            pl.BlockSpec((_BN, D), lambda i: (i, 0)),
            pl.BlockSpec((_BN, D), lambda i: (i, 0)),
        ],
        out_shape=[
            jax.ShapeDtypeStruct((N, D), jnp.float32),
            jax.ShapeDtypeStruct((N, D), jnp.float32),
        ],
    )(x, w0t, w1t, b)


def _tc_mid_body(a_ref, agg_ref, w0t_ref, w1t_ref, b_ref, a_out_ref, z_out_ref):
    h = a_ref[...] + agg_ref[0] + agg_ref[1]
    y = jnp.where(h >= 0, h, _slope * h)
    a_out_ref[...] = _dot_t(y, w0t_ref[...]) + b_ref[...] + y
    z_out_ref[...] = _dot_t(y, w1t_ref[...])


def _tc_mid(a, agg, w0t, w1t, b):
    grid = (N // _BN,)
    return pl.pallas_call(
        _tc_mid_body,
        grid=grid,
        in_specs=[
            pl.BlockSpec((_BN, D), lambda i: (i, 0)),
            pl.BlockSpec((NC, _BN, D), lambda i: (0, i, 0)),
            pl.BlockSpec((D, D), lambda i: (0, 0)),
            pl.BlockSpec((D, D), lambda i: (0, 0)),
            pl.BlockSpec((1, D), lambda i: (0, 0)),
        ],
        out_specs=[
            pl.BlockSpec((_BN, D), lambda i: (i, 0)),
            pl.BlockSpec((_BN, D), lambda i: (i, 0)),
        ],
        out_shape=[
            jax.ShapeDtypeStruct((N, D), jnp.float32),
            jax.ShapeDtypeStruct((N, D), jnp.float32),
        ],
    )(a, agg, w0t, w1t, b)


def _tc_post_body(a_ref, agg_ref, y_ref):
    h = a_ref[...] + agg_ref[0] + agg_ref[1]
    y_ref[...] = jnp.where(h >= 0, h, _slope * h)


def _tc_post(a, agg):
    grid = (N // _BN,)
    return pl.pallas_call(
        _tc_post_body,
        grid=grid,
        in_specs=[
            pl.BlockSpec((_BN, D), lambda i: (i, 0)),
            pl.BlockSpec((NC, _BN, D), lambda i: (0, i, 0)),
        ],
        out_specs=pl.BlockSpec((_BN, D), lambda i: (i, 0)),
        out_shape=jax.ShapeDtypeStruct((N, D), jnp.float32),
    )(a, agg)


def kernel(y, edge_index, edge_weight, W0_0, W1_0, b_0, W0_1, W1_1, b_1):
    a0, z0 = _tc_pre(y, W0_0, W1_0, b_0.reshape(1, D))
    agg0 = _sc_agg(z0, edge_index, edge_weight)
    a1, z1 = _tc_mid(a0, agg0, W0_1, W1_1, b_1.reshape(1, D))
    agg1 = _sc_agg(z1, edge_index, edge_weight)
    return _tc_post(a1, agg1)


# speedup vs baseline: 3.6120x; 1.0532x over previous
import functools

import jax
import jax.numpy as jnp
from jax import lax
from jax.experimental import pallas as pl
from jax.experimental.pallas import tpu as pltpu
from jax.experimental.pallas import tpu_sc as plsc

N = 10000
E = 320000
D = 128

NC = 2
NS = 16
NW = NC * NS

C = 128
G = E // C
CPW = G // NW
NEXTRA = G - NW * CPW
NP = N

_slope = 0.01

_GATHER_DNUMS = lax.GatherDimensionNumbers(
    offset_dims=(), collapsed_slice_dims=(0,), start_index_map=(0,))


def _lane_bcast(vec, lane):
    idx = jnp.full((16, 1), lane, jnp.int32)
    return lax.gather(vec, idx, dimension_numbers=_GATHER_DNUMS,
                      slice_sizes=(1,),
                      mode=lax.GatherScatterMode.PROMISE_IN_BOUNDS)


def _sc_agg_body(z_hbm, ei_hbm, ew_hbm, out_hbm,
                 src0, src1, src2, dst0, dst1, ew0, ew1, ew2,
                 rows0, rows1, rows2, acc_sh,
                 sem_e0, sem_e1, sem_e2, sem_d0, sem_d1,
                 sem_g0, sem_g1, sem_g2, sem_s0, sem_s1, sem_s2):
    cid = lax.axis_index("c")
    sid = lax.axis_index("s")
    wid = sid * NC + cid

    srcs = (src0, src1, src2)
    dsts = (dst0, dst1)
    ews = (ew0, ew1, ew2)
    rows = (rows0, rows1, rows2)
    sem_e = (sem_e0, sem_e1, sem_e2)
    sem_d = (sem_d0, sem_d1)
    sem_g = (sem_g0, sem_g1, sem_g2)
    sem_s = (sem_s0, sem_s1, sem_s2)

    def _eb(k):
        return (wid + k * NW) * C

    def _se_start(k, p3):
        eb = _eb(k)
        pltpu.async_copy(ei_hbm.at[0, pl.ds(eb, C)], srcs[p3], sem_e[p3])
        pltpu.async_copy(ew_hbm.at[pl.ds(eb, C)], ews[p3], sem_e[p3])

    def _se_wait(k, p3):
        eb = _eb(k)
        pltpu.make_async_copy(ei_hbm.at[0, pl.ds(eb, C)], srcs[p3], sem_e[p3]).wait()
        pltpu.make_async_copy(ew_hbm.at[pl.ds(eb, C)], ews[p3], sem_e[p3]).wait()

    def _dst_start(k, pd):
        pltpu.async_copy(ei_hbm.at[1, pl.ds(_eb(k), C)], dsts[pd], sem_d[pd])

    def _dst_wait(k, pd):
        pltpu.make_async_copy(ei_hbm.at[1, pl.ds(_eb(k), C)], dsts[pd],
                              sem_d[pd]).wait()

    def _gather(p3):
        pltpu.async_copy(z_hbm.at[srcs[p3]], rows[p3], sem_g[p3])

    def _gwait(p3):
        pltpu.make_async_copy(z_hbm.at[srcs[0]], rows[p3], sem_g[p3]).wait()

    def _scale(p3):
        ew_v = ews[p3]
        buf = rows[p3]

        def _g(g, _):
            wvec = ew_v[pl.ds(g * 16, 16)]
            for lane in range(16):
                bw = _lane_bcast(wvec, lane)
                e = g * 16 + lane
                for j in range(D // 16):
                    sl = pl.ds(j * 16, 16)
                    buf[e, sl] = buf[e, sl] * bw
            return 0
        lax.fori_loop(0, C // 16, _g, 0)

    def _sstart(p3, pd):
        pltpu.async_copy(rows[p3], acc_sh.at[dsts[pd]], sem_s[p3], add=True)

    def _swait(p3):
        pltpu.make_async_copy(rows[p3], acc_sh.at[dsts[0]], sem_s[p3]).wait()

    _se_start(0, 0)
    _se_start(1, 1)
    _se_start(2, 2)
    _dst_start(0, 0)

    def _zrow(i, _):
        for j in range(D // 16):
            rows0[i, pl.ds(j * 16, 16)] = jnp.zeros((16,), jnp.float32)
        return 0
    lax.fori_loop(0, C, _zrow, 0)
    zbase = sid * 632
    zoffs = [jnp.minimum(zbase + kz * C, NP - C) for kz in range(5)]
    for zo in zoffs:
        pltpu.sync_copy(rows0, acc_sh.at[pl.ds(zo, C)])
    plsc.subcore_barrier()

    _se_wait(0, 0)
    _gather(0)
    _se_wait(1, 1)
    _gather(1)

    def _iter(k, p3, pd):
        k = jnp.asarray(k, jnp.int32)

        @pl.when(k >= 1)
        def _():
            _swait((p3 + 2) % 3)

        @pl.when(k < CPW - 1)
        def _():
            _dst_start(k + 1, 1 - pd)

        @pl.when(k < CPW - 2)
        def _():
            _se_wait(k + 2, (p3 + 2) % 3)
            _gather((p3 + 2) % 3)
        _gwait(p3)
        _dst_wait(k, pd)
        _scale(p3)
        _sstart(p3, pd)

        @pl.when(k < CPW - 3)
        def _():
            _se_start(k + 3, p3)

    def _iter6(i6, _):
        k0 = 6 * i6
        _iter(k0, 0, 0)
        _iter(k0 + 1, 1, 1)
        _iter(k0 + 2, 2, 0)
        _iter(k0 + 3, 0, 1)
        _iter(k0 + 4, 1, 0)
        _iter(k0 + 5, 2, 1)
        return 0
    lax.fori_loop(0, CPW // 6 - 1, _iter6, 0)
    for k in range(6 * (CPW // 6 - 1), CPW):
        _iter(k, k % 3, k % 2)
    _swait((CPW - 1) % 3)

    @pl.when(wid < NEXTRA)
    def _():
        eb = (NW * CPW + wid) * C
        pltpu.sync_copy(ei_hbm.at[0, pl.ds(eb, C)], srcs[0])
        pltpu.sync_copy(ei_hbm.at[1, pl.ds(eb, C)], dsts[0])
        pltpu.sync_copy(ew_hbm.at[pl.ds(eb, C)], ews[0])
        pltpu.async_copy(z_hbm.at[srcs[0]], rows[0], sem_g[0]).wait()
        _scale(0)
        pltpu.sync_copy(rows[0], acc_sh.at[dsts[0]], add=True)

    plsc.subcore_barrier()

    for zo in zoffs:
        pltpu.sync_copy(acc_sh.at[pl.ds(zo, C)],
                        out_hbm.at[cid, pl.ds(zo, C)])


@functools.partial(jax.jit, static_argnames=())
def _sc_agg(z, ei, ew):
    mesh = plsc.VectorSubcoreMesh(core_axis_name="c", subcore_axis_name="s")
    f = pl.kernel(
        _sc_agg_body,
        out_type=jax.ShapeDtypeStruct((NC, NP, D), jnp.float32),
        mesh=mesh,
        scratch_types=(
            [pltpu.VMEM((C,), jnp.int32)] * 5
            + [pltpu.VMEM((C,), jnp.float32)] * 3
            + [pltpu.VMEM((C, D), jnp.float32)] * 3
            + [pltpu.VMEM_SHARED((NP, D), jnp.float32)]
            + [pltpu.SemaphoreType.DMA] * 11
        ),
    )
    return f(z, ei, ew)


_BN = 1000


def _dot_t(x, w):
    return lax.dot_general(x, w, (((1,), (1,)), ((), ())),
                           preferred_element_type=jnp.float32)


def _tc_pre_body(x_ref, w0t_ref, w1t_ref, b_ref, a_ref, z_ref):
    x = x_ref[...]
    a_ref[...] = _dot_t(x, w0t_ref[...]) + b_ref[...] + x
    z_ref[...] = _dot_t(x, w1t_ref[...])


def _tc_pre(x, w0t, w1t, b):
    grid = (N // _BN,)
    return pl.pallas_call(
        _tc_pre_body,
        grid=grid,
        in_specs=[
            pl.BlockSpec((_BN, D), lambda i: (i, 0)),
            pl.BlockSpec((D, D), lambda i: (0, 0)),
            pl.BlockSpec((D, D), lambda i: (0, 0)),
            pl.BlockSpec((1, D), lambda i: (0, 0)),
        ],
        out_specs=[
            pl.BlockSpec((_BN, D), lambda i: (i, 0)),
            pl.BlockSpec((_BN, D), lambda i: (i, 0)),
        ],
        out_shape=[
            jax.ShapeDtypeStruct((N, D), jnp.float32),
            jax.ShapeDtypeStruct((N, D), jnp.float32),
        ],
    )(x, w0t, w1t, b)


def _tc_mid_body(a_ref, agg_ref, w0t_ref, w1t_ref, b_ref, a_out_ref, z_out_ref):
    h = a_ref[...] + agg_ref[0] + agg_ref[1]
    y = jnp.where(h >= 0, h, _slope * h)
    a_out_ref[...] = _dot_t(y, w0t_ref[...]) + b_ref[...] + y
    z_out_ref[...] = _dot_t(y, w1t_ref[...])


def _tc_mid(a, agg, w0t, w1t, b):
    grid = (N // _BN,)
    return pl.pallas_call(
        _tc_mid_body,
        grid=grid,
        in_specs=[
            pl.BlockSpec((_BN, D), lambda i: (i, 0)),
            pl.BlockSpec((NC, _BN, D), lambda i: (0, i, 0)),
            pl.BlockSpec((D, D), lambda i: (0, 0)),
            pl.BlockSpec((D, D), lambda i: (0, 0)),
            pl.BlockSpec((1, D), lambda i: (0, 0)),
        ],
        out_specs=[
            pl.BlockSpec((_BN, D), lambda i: (i, 0)),
            pl.BlockSpec((_BN, D), lambda i: (i, 0)),
        ],
        out_shape=[
            jax.ShapeDtypeStruct((N, D), jnp.float32),
            jax.ShapeDtypeStruct((N, D), jnp.float32),
        ],
    )(a, agg, w0t, w1t, b)


def _tc_post_body(a_ref, agg_ref, y_ref):
    h = a_ref[...] + agg_ref[0] + agg_ref[1]
    y_ref[...] = jnp.where(h >= 0, h, _slope * h)


def _tc_post(a, agg):
    grid = (N // _BN,)
    return pl.pallas_call(
        _tc_post_body,
        grid=grid,
        in_specs=[
            pl.BlockSpec((_BN, D), lambda i: (i, 0)),
            pl.BlockSpec((NC, _BN, D), lambda i: (0, i, 0)),
        ],
        out_specs=pl.BlockSpec((_BN, D), lambda i: (i, 0)),
        out_shape=jax.ShapeDtypeStruct((N, D), jnp.float32),
    )(a, agg)


def kernel(y, edge_index, edge_weight, W0_0, W1_0, b_0, W0_1, W1_1, b_1):
    a0, z0 = _tc_pre(y, W0_0, W1_0, b_0.reshape(1, D))
    agg0 = _sc_agg(z0, edge_index, edge_weight)
    a1, z1 = _tc_mid(a0, agg0, W0_1, W1_1, b_1.reshape(1, D))
    agg1 = _sc_agg(z1, edge_index, edge_weight)
    return _tc_post(a1, agg1)


# speedup vs baseline: 3.6618x; 1.0138x over previous
import functools

import jax
import jax.numpy as jnp
from jax import lax
from jax.experimental import pallas as pl
from jax.experimental.pallas import tpu as pltpu
from jax.experimental.pallas import tpu_sc as plsc

N = 10000
E = 320000
D = 128

NC = 2
NS = 16
NW = NC * NS

C = 128
G = E // C
CPW = G // NW
NEXTRA = G - NW * CPW
NP = N

_slope = 0.01

_GATHER_DNUMS = lax.GatherDimensionNumbers(
    offset_dims=(), collapsed_slice_dims=(0,), start_index_map=(0,))


def _lane_bcast(vec, lane):
    idx = jnp.full((16, 1), lane, jnp.int32)
    return lax.gather(vec, idx, dimension_numbers=_GATHER_DNUMS,
                      slice_sizes=(1,),
                      mode=lax.GatherScatterMode.PROMISE_IN_BOUNDS)


def _sc_agg_body(z_hbm, ei_hbm, ew_hbm, out_hbm,
                 src0, src1, src2, dst0, dst1, ew0, ew1, ew2,
                 rows0, rows1, rows2, acc_sh,
                 sem_e0, sem_e1, sem_e2, sem_d0, sem_d1,
                 sem_g0, sem_g1, sem_g2, sem_s0, sem_s1, sem_s2):
    cid = lax.axis_index("c")
    sid = lax.axis_index("s")
    wid = sid * NC + cid

    srcs = (src0, src1, src2)
    dsts = (dst0, dst1)
    ews = (ew0, ew1, ew2)
    rows = (rows0, rows1, rows2)
    sem_e = (sem_e0, sem_e1, sem_e2)
    sem_d = (sem_d0, sem_d1)
    sem_g = (sem_g0, sem_g1, sem_g2)
    sem_s = (sem_s0, sem_s1, sem_s2)

    def _eb(k):
        return (wid + k * NW) * C

    def _se_start(k, p3):
        eb = _eb(k)
        pltpu.async_copy(ei_hbm.at[0, pl.ds(eb, C)], srcs[p3], sem_e[p3])
        pltpu.async_copy(ew_hbm.at[pl.ds(eb, C)], ews[p3], sem_e[p3])

    def _se_wait(k, p3):
        eb = _eb(k)
        pltpu.make_async_copy(ei_hbm.at[0, pl.ds(eb, C)], srcs[p3], sem_e[p3]).wait()
        pltpu.make_async_copy(ew_hbm.at[pl.ds(eb, C)], ews[p3], sem_e[p3]).wait()

    def _dst_start(k, pd):
        pltpu.async_copy(ei_hbm.at[1, pl.ds(_eb(k), C)], dsts[pd], sem_d[pd])

    def _dst_wait(k, pd):
        pltpu.make_async_copy(ei_hbm.at[1, pl.ds(_eb(k), C)], dsts[pd],
                              sem_d[pd]).wait()

    def _gather(p3):
        pltpu.async_copy(z_hbm.at[srcs[p3]], rows[p3], sem_g[p3])

    def _gwait(p3):
        pltpu.make_async_copy(z_hbm.at[srcs[0]], rows[p3], sem_g[p3]).wait()

    def _scale(p3):
        ew_v = ews[p3]
        buf = rows[p3]

        def _g(g, _):
            wvec = ew_v[pl.ds(g * 16, 16)]
            for lane in range(16):
                bw = _lane_bcast(wvec, lane)
                e = g * 16 + lane
                for j in range(D // 16):
                    sl = pl.ds(j * 16, 16)
                    buf[e, sl] = buf[e, sl] * bw
            return 0
        lax.fori_loop(0, C // 16, _g, 0)

    def _sstart(p3, pd):
        pltpu.async_copy(rows[p3], acc_sh.at[dsts[pd]], sem_s[p3], add=True)

    def _swait(p3):
        pltpu.make_async_copy(rows[p3], acc_sh.at[dsts[0]], sem_s[p3]).wait()

    _se_start(0, 0)
    _se_start(1, 1)
    _se_start(2, 2)
    _dst_start(0, 0)

    def _zrow(i, _):
        for j in range(D // 16):
            rows0[i, pl.ds(j * 16, 16)] = jnp.zeros((16,), jnp.float32)
        return 0
    lax.fori_loop(0, C, _zrow, 0)
    zbase = sid * 632
    zoffs = [jnp.minimum(zbase + kz * C, NP - C) for kz in range(5)]
    for zo in zoffs:
        pltpu.sync_copy(rows0, acc_sh.at[pl.ds(zo, C)])
    plsc.subcore_barrier()

    _se_wait(0, 0)
    _gather(0)
    _se_wait(1, 1)
    _gather(1)

    def _iter(k, p3, pd, do_swait=True, do_dst=True, do_gather=True,
              do_se=True):
        if do_swait:
            _swait((p3 + 2) % 3)
        if do_dst:
            _dst_start(k + 1, 1 - pd)
        if do_gather:
            _se_wait(k + 2, (p3 + 2) % 3)
            _gather((p3 + 2) % 3)
        _gwait(p3)
        _dst_wait(k, pd)
        _scale(p3)
        _sstart(p3, pd)
        if do_se:
            _se_start(k + 3, p3)

    for k in range(6):
        _iter(k, k % 3, k % 2, do_swait=(k >= 1))

    def _iter6(i6, _):
        k0 = 6 * i6
        _iter(k0, 0, 0)
        _iter(k0 + 1, 1, 1)
        _iter(k0 + 2, 2, 0)
        _iter(k0 + 3, 0, 1)
        _iter(k0 + 4, 1, 0)
        _iter(k0 + 5, 2, 1)
        return 0
    lax.fori_loop(1, CPW // 6 - 1, _iter6, 0)
    for k in range(6 * (CPW // 6 - 1), CPW):
        _iter(k, k % 3, k % 2, do_dst=(k < CPW - 1),
              do_gather=(k < CPW - 2), do_se=(k < CPW - 3))
    _swait((CPW - 1) % 3)

    @pl.when(wid < NEXTRA)
    def _():
        eb = (NW * CPW + wid) * C
        pltpu.sync_copy(ei_hbm.at[0, pl.ds(eb, C)], srcs[0])
        pltpu.sync_copy(ei_hbm.at[1, pl.ds(eb, C)], dsts[0])
        pltpu.sync_copy(ew_hbm.at[pl.ds(eb, C)], ews[0])
        pltpu.async_copy(z_hbm.at[srcs[0]], rows[0], sem_g[0]).wait()
        _scale(0)
        pltpu.sync_copy(rows[0], acc_sh.at[dsts[0]], add=True)

    plsc.subcore_barrier()

    for zo in zoffs:
        pltpu.sync_copy(acc_sh.at[pl.ds(zo, C)],
                        out_hbm.at[cid, pl.ds(zo, C)])


@functools.partial(jax.jit, static_argnames=())
def _sc_agg(z, ei, ew):
    mesh = plsc.VectorSubcoreMesh(core_axis_name="c", subcore_axis_name="s")
    f = pl.kernel(
        _sc_agg_body,
        out_type=jax.ShapeDtypeStruct((NC, NP, D), jnp.float32),
        mesh=mesh,
        scratch_types=(
            [pltpu.VMEM((C,), jnp.int32)] * 5
            + [pltpu.VMEM((C,), jnp.float32)] * 3
            + [pltpu.VMEM((C, D), jnp.float32)] * 3
            + [pltpu.VMEM_SHARED((NP, D), jnp.float32)]
            + [pltpu.SemaphoreType.DMA] * 11
        ),
    )
    return f(z, ei, ew)


_BN = 2000


def _dot_t(x, w):
    return lax.dot_general(x, w, (((1,), (1,)), ((), ())),
                           preferred_element_type=jnp.float32)


def _tc_pre_body(x_ref, w0t_ref, w1t_ref, b_ref, a_ref, z_ref):
    x = x_ref[...]
    a_ref[...] = _dot_t(x, w0t_ref[...]) + b_ref[...] + x
    z_ref[...] = _dot_t(x, w1t_ref[...])


def _tc_pre(x, w0t, w1t, b):
    grid = (N // _BN,)
    return pl.pallas_call(
        _tc_pre_body,
        grid=grid,
        in_specs=[
            pl.BlockSpec((_BN, D), lambda i: (i, 0)),
            pl.BlockSpec((D, D), lambda i: (0, 0)),
            pl.BlockSpec((D, D), lambda i: (0, 0)),
            pl.BlockSpec((1, D), lambda i: (0, 0)),
        ],
        out_specs=[
            pl.BlockSpec((_BN, D), lambda i: (i, 0)),
            pl.BlockSpec((_BN, D), lambda i: (i, 0)),
        ],
        out_shape=[
            jax.ShapeDtypeStruct((N, D), jnp.float32),
            jax.ShapeDtypeStruct((N, D), jnp.float32),
        ],
    )(x, w0t, w1t, b)


def _tc_mid_body(a_ref, agg_ref, w0t_ref, w1t_ref, b_ref, a_out_ref, z_out_ref):
    h = a_ref[...] + agg_ref[0] + agg_ref[1]
    y = jnp.where(h >= 0, h, _slope * h)
    a_out_ref[...] = _dot_t(y, w0t_ref[...]) + b_ref[...] + y
    z_out_ref[...] = _dot_t(y, w1t_ref[...])


def _tc_mid(a, agg, w0t, w1t, b):
    grid = (N // _BN,)
    return pl.pallas_call(
        _tc_mid_body,
        grid=grid,
        in_specs=[
            pl.BlockSpec((_BN, D), lambda i: (i, 0)),
            pl.BlockSpec((NC, _BN, D), lambda i: (0, i, 0)),
            pl.BlockSpec((D, D), lambda i: (0, 0)),
            pl.BlockSpec((D, D), lambda i: (0, 0)),
            pl.BlockSpec((1, D), lambda i: (0, 0)),
        ],
        out_specs=[
            pl.BlockSpec((_BN, D), lambda i: (i, 0)),
            pl.BlockSpec((_BN, D), lambda i: (i, 0)),
        ],
        out_shape=[
            jax.ShapeDtypeStruct((N, D), jnp.float32),
            jax.ShapeDtypeStruct((N, D), jnp.float32),
        ],
    )(a, agg, w0t, w1t, b)


def _tc_post_body(a_ref, agg_ref, y_ref):
    h = a_ref[...] + agg_ref[0] + agg_ref[1]
    y_ref[...] = jnp.where(h >= 0, h, _slope * h)


def _tc_post(a, agg):
    grid = (N // _BN,)
    return pl.pallas_call(
        _tc_post_body,
        grid=grid,
        in_specs=[
            pl.BlockSpec((_BN, D), lambda i: (i, 0)),
            pl.BlockSpec((NC, _BN, D), lambda i: (0, i, 0)),
        ],
        out_specs=pl.BlockSpec((_BN, D), lambda i: (i, 0)),
        out_shape=jax.ShapeDtypeStruct((N, D), jnp.float32),
    )(a, agg)


def kernel(y, edge_index, edge_weight, W0_0, W1_0, b_0, W0_1, W1_1, b_1):
    a0, z0 = _tc_pre(y, W0_0, W1_0, b_0.reshape(1, D))
    agg0 = _sc_agg(z0, edge_index, edge_weight)
    a1, z1 = _tc_mid(a0, agg0, W0_1, W1_1, b_1.reshape(1, D))
    agg1 = _sc_agg(z1, edge_index, edge_weight)
    return _tc_post(a1, agg1)
